# baseline (speedup 1.0000x reference)
import functools

import jax
import jax.numpy as jnp
from jax import lax
from jax.experimental import pallas as pl
from jax.experimental.pallas import tpu as pltpu
from jax.experimental.pallas import tpu_sc as plsc

_CHUNK = 128


def _gather_body(table_hbm, idx_hbm, out_hbm, idx_v, rows_v, sem, *, nc, b_per_w):
    wid = lax.axis_index("s") * nc + lax.axis_index("c")
    n_chunks = b_per_w // _CHUNK
    pltpu.sync_copy(idx_hbm.at[wid], idx_v)
    copies = []
    for j in range(n_chunks):
        copies.append(
            pltpu.async_copy(
                table_hbm.at[idx_v.at[j]],
                rows_v.at[pl.ds(j * _CHUNK, _CHUNK)],
                sem,
            )
        )
    for c in copies:
        c.wait()
    pltpu.sync_copy(rows_v, out_hbm.at[pl.ds(wid * b_per_w, b_per_w)])


def kernel(positional_encoding, time_steps):
    V, D = positional_encoding.shape
    (B,) = time_steps.shape
    info = plsc.get_sparse_core_info()
    nc, ns = info.num_cores, info.num_subcores
    nw = nc * ns
    b_per_w = B // nw
    n_chunks = b_per_w // _CHUNK
    idx3 = time_steps.reshape(nw, n_chunks, _CHUNK)
    mesh = plsc.VectorSubcoreMesh(core_axis_name="c", subcore_axis_name="s")
    run = pl.kernel(
        functools.partial(_gather_body, nc=nc, b_per_w=b_per_w),
        mesh=mesh,
        out_type=jax.ShapeDtypeStruct((B, D), positional_encoding.dtype),
        scratch_types=[
            pltpu.VMEM((n_chunks, _CHUNK), jnp.int32),
            pltpu.VMEM((b_per_w, D), jnp.float32),
            pltpu.SemaphoreType.DMA,
        ],
        compiler_params=pltpu.CompilerParams(use_tc_tiling_on_sc=False),
    )
    return run(positional_encoding, idx3)


# speedup vs baseline: 1.0016x; 1.0016x over previous
import functools

import jax
import jax.numpy as jnp
from jax import lax
from jax.experimental import pallas as pl
from jax.experimental.pallas import tpu as pltpu
from jax.experimental.pallas import tpu_sc as plsc

_CHUNK = 128


def _gather_body(table_hbm, idx_hbm, out_hbm, idx_v, rows_v, sem, *, nc, b_per_w):
    wid = lax.axis_index("s") * nc + lax.axis_index("c")
    base = wid * b_per_w
    n_chunks = b_per_w // _CHUNK
    pltpu.sync_copy(idx_hbm.at[pl.ds(base, b_per_w)], idx_v)
    copies = []
    for j in range(n_chunks):
        copies.append(
            pltpu.async_copy(
                table_hbm.at[idx_v.at[pl.ds(j * _CHUNK, _CHUNK)]],
                rows_v.at[pl.ds(j * _CHUNK, _CHUNK)],
                sem,
            )
        )
    for c in copies:
        c.wait()
    pltpu.sync_copy(rows_v, out_hbm.at[pl.ds(base, b_per_w)])


def kernel(positional_encoding, time_steps):
    V, D = positional_encoding.shape
    (B,) = time_steps.shape
    info = plsc.get_sparse_core_info()
    nc, ns = info.num_cores, info.num_subcores
    nw = nc * ns
    b_per_w = B // nw
    mesh = plsc.VectorSubcoreMesh(core_axis_name="c", subcore_axis_name="s")
    run = pl.kernel(
        functools.partial(_gather_body, nc=nc, b_per_w=b_per_w),
        mesh=mesh,
        out_type=jax.ShapeDtypeStruct((B, D), positional_encoding.dtype),
        scratch_types=[
            pltpu.VMEM((b_per_w,), jnp.int32),
            pltpu.VMEM((b_per_w, D), jnp.float32),
            pltpu.SemaphoreType.DMA,
        ],
        compiler_params=pltpu.CompilerParams(use_tc_tiling_on_sc=False),
    )
    return run(positional_encoding, time_steps)
